# baseline (speedup 1.0000x reference)
import functools

import jax
import jax.numpy as jnp
from jax import lax
from jax.experimental import pallas as pl
from jax.experimental.pallas import tpu as pltpu
from jax.experimental.pallas import tpu_sc as plsc

_NC = 2
_NS = 16
_NW = _NC * _NS


def _pick_chunk(per_worker: int) -> int:
    for c in range(min(per_worker // 2, 8184), 0, -8):
        if per_worker % c == 0 and (per_worker // c) % 2 == 0:
            return c
    raise ValueError(f"cannot chunk {per_worker}")


@functools.lru_cache(maxsize=None)
def _build(bl: int):
    assert bl % _NW == 0, bl
    per_w = bl // _NW
    chunk = _pick_chunk(per_w)
    n_steps = per_w // chunk
    assert n_steps >= 4 and n_steps % 2 == 0, n_steps

    mesh = plsc.VectorSubcoreMesh(
        core_axis_name="c", subcore_axis_name="s",
        num_cores=_NC, num_subcores=_NS)

    scratch = []
    for _ in range(2):
        scratch += [pltpu.VMEM((chunk,), jnp.int32)] * 4
        scratch += [pltpu.VMEM((chunk,), jnp.float32)] * 4
    scratch += [pltpu.SemaphoreType.DMA] * 6

    @functools.partial(
        pl.kernel,
        out_type=jax.ShapeDtypeStruct((4, bl), jnp.float32),
        mesh=mesh,
        scratch_types=scratch,
    )
    def gather_kernel(tc_h, mag_h, tia_h, tib_h, mia_h, mib_h, out_h,
                      p0, p1, p2, p3, a0, a1, a2, a3,
                      q0, q1, q2, q3, b0, b1, b2, b3,
                      isem0, gsem0, ssem0, isem1, gsem1, ssem1):
        wid = lax.axis_index("s") * _NC + lax.axis_index("c")
        w_base = wid * per_w
        idx_hbm = (tia_h, tib_h, mia_h, mib_h)
        tabs = (tc_h, tc_h, mag_h, mag_h)
        bufs = (((p0, p1, p2, p3), (a0, a1, a2, a3), isem0, gsem0, ssem0),
                ((q0, q1, q2, q3), (b0, b1, b2, b3), isem1, gsem1, ssem1))

        def fire_idx(s, b):
            iv, _, isem, _, _ = bufs[b]
            base = w_base + s * chunk
            for t in range(4):
                pltpu.async_copy(idx_hbm[t].at[pl.ds(base, chunk)],
                                 iv[t], isem)

        def drain_idx(b):
            iv, _, isem, _, _ = bufs[b]
            for t in range(4):
                pltpu.make_async_copy(
                    tia_h.at[pl.ds(0, chunk)], iv[t], isem).wait()

        def fire_gathers(b):
            iv, d, _, gsem, _ = bufs[b]
            for t in range(4):
                pltpu.async_copy(tabs[t].at[iv[t]], d[t], gsem)

        def drain_gathers(b):
            _, d, _, gsem, _ = bufs[b]
            for t in range(4):
                pltpu.make_async_copy(
                    tc_h.at[pl.ds(0, chunk)], d[t], gsem).wait()

        def fire_stores(s, b):
            _, d, _, _, ssem = bufs[b]
            base = w_base + s * chunk
            for t in range(4):
                pltpu.async_copy(d[t], out_h.at[t, pl.ds(base, chunk)], ssem)

        def drain_stores(b):
            _, d, _, _, ssem = bufs[b]
            for t in range(4):
                pltpu.make_async_copy(
                    tc_h.at[pl.ds(0, chunk)], d[t], ssem).wait()

        def body(s, b):
            drain_idx(b)
            drain_stores(b)
            fire_gathers(b)
            drain_gathers(1 - b)
            fire_stores(s - 1, 1 - b)
            return s + 1

        fire_idx(0, 0)
        fire_idx(1, 1)
        drain_idx(0)
        fire_gathers(0)
        drain_idx(1)
        fire_gathers(1)
        drain_gathers(0)
        fire_stores(0, 0)
        fire_idx(2, 0)

        def loop_body(u, c):
            s = 2 * u
            fire_idx(body(s, 0), 1)
            fire_idx(body(s + 1, 1), 0)
            return c
        lax.fori_loop(1, n_steps // 2 - 1, loop_body, 0)

        fire_idx(body(n_steps - 2, 0), 1)
        body(n_steps - 1, 1)

        drain_gathers(1)
        fire_stores(n_steps - 1, 1)
        drain_stores(0)
        drain_stores(1)

    return gather_kernel


def kernel(theta_indices, mag_indices, cos_table, sin_table, exp_table,
           exp_deriv_table):
    b, l = theta_indices.shape
    bl = b * l
    ti = theta_indices.reshape(bl).astype(jnp.int32)
    mi = mag_indices.reshape(bl).astype(jnp.int32)
    tia = ti * 2
    tib = tia + 1
    mia = mi * 2
    mib = mia + 1
    tp_tc = jnp.stack([cos_table, jnp.negative(sin_table)], axis=1).reshape(-1)
    tp_mag = jnp.stack([exp_table, exp_deriv_table], axis=1).reshape(-1)
    out = _build(bl)(tp_tc, tp_mag, tia, tib, mia, mib)
    return out.reshape(4, b, l)


# speedup vs baseline: 2.5278x; 2.5278x over previous
import functools

import jax
import jax.numpy as jnp
from jax import lax
from jax.experimental import pallas as pl
from jax.experimental.pallas import tpu as pltpu
from jax.experimental.pallas import tpu_sc as plsc

_NC = 2
_NS = 16
_NW = _NC * _NS


def _pick_chunk(per_worker: int) -> int:
    for c in range(min(per_worker // 2, 10240), 0, -8):
        if per_worker % c == 0 and (per_worker // c) % 2 == 0:
            return c
    raise ValueError(f"cannot chunk {per_worker}")


@functools.lru_cache(maxsize=None)
def _build(bl: int):
    assert bl % _NW == 0, bl
    per_w = bl // _NW
    chunk = _pick_chunk(per_w)
    n_steps = per_w // chunk
    assert n_steps >= 4 and n_steps % 2 == 0, n_steps

    mesh = plsc.VectorSubcoreMesh(
        core_axis_name="c", subcore_axis_name="s",
        num_cores=_NC, num_subcores=_NS)

    scratch = []
    for _ in range(2):
        scratch += [pltpu.VMEM((chunk,), jnp.int32)] * 2
        scratch += [pltpu.VMEM((chunk,), jnp.float32)] * 4
    scratch += [pltpu.SemaphoreType.DMA] * 6

    @functools.partial(
        pl.kernel,
        out_type=jax.ShapeDtypeStruct((4, bl), jnp.float32),
        mesh=mesh,
        scratch_types=scratch,
    )
    def gather_kernel(cos_h, nsin_h, exp_h, expd_h, ti_h, mi_h, out_h,
                      it0, im0, a0, a1, a2, a3,
                      it1, im1, b0, b1, b2, b3,
                      isem0, gsem0, ssem0, isem1, gsem1, ssem1):
        wid = lax.axis_index("s") * _NC + lax.axis_index("c")
        w_base = wid * per_w
        tabs = (cos_h, nsin_h, exp_h, expd_h)
        bufs = ((it0, im0, (a0, a1, a2, a3), isem0, gsem0, ssem0),
                (it1, im1, (b0, b1, b2, b3), isem1, gsem1, ssem1))

        def fire_idx(s, b):
            it_v, im_v, _, isem, _, _ = bufs[b]
            base = w_base + s * chunk
            pltpu.async_copy(ti_h.at[pl.ds(base, chunk)], it_v, isem)
            pltpu.async_copy(mi_h.at[pl.ds(base, chunk)], im_v, isem)

        def drain_idx(b):
            it_v, im_v, _, isem, _, _ = bufs[b]
            pltpu.make_async_copy(ti_h.at[pl.ds(0, chunk)], it_v, isem).wait()
            pltpu.make_async_copy(mi_h.at[pl.ds(0, chunk)], im_v, isem).wait()

        def fire_gathers(b):
            it_v, im_v, d, _, gsem, _ = bufs[b]
            idxs = (it_v, it_v, im_v, im_v)
            for t in range(4):
                pltpu.async_copy(tabs[t].at[idxs[t]], d[t], gsem)

        def drain_gathers(b):
            _, _, d, _, gsem, _ = bufs[b]
            for t in range(4):
                pltpu.make_async_copy(
                    cos_h.at[pl.ds(0, chunk)], d[t], gsem).wait()

        def fire_stores(s, b):
            _, _, d, _, _, ssem = bufs[b]
            base = w_base + s * chunk
            for t in range(4):
                pltpu.async_copy(d[t], out_h.at[t, pl.ds(base, chunk)], ssem)

        def drain_stores(b):
            _, _, d, _, _, ssem = bufs[b]
            for t in range(4):
                pltpu.make_async_copy(
                    cos_h.at[pl.ds(0, chunk)], d[t], ssem).wait()

        def body(s, b):
            drain_idx(b)
            drain_stores(b)
            fire_gathers(b)
            drain_gathers(1 - b)
            fire_stores(s - 1, 1 - b)
            return s + 1

        fire_idx(0, 0)
        fire_idx(1, 1)
        drain_idx(0)
        fire_gathers(0)
        drain_idx(1)
        fire_gathers(1)
        drain_gathers(0)
        fire_stores(0, 0)
        fire_idx(2, 0)

        def loop_body(u, c):
            s = 2 * u
            fire_idx(body(s, 0), 1)
            fire_idx(body(s + 1, 1), 0)
            return c
        lax.fori_loop(1, n_steps // 2 - 1, loop_body, 0)

        fire_idx(body(n_steps - 2, 0), 1)
        body(n_steps - 1, 1)

        drain_gathers(1)
        fire_stores(n_steps - 1, 1)
        drain_stores(0)
        drain_stores(1)

    return gather_kernel


def kernel(theta_indices, mag_indices, cos_table, sin_table, exp_table,
           exp_deriv_table):
    b, l = theta_indices.shape
    bl = b * l
    ti = theta_indices.reshape(bl).astype(jnp.int32)
    mi = mag_indices.reshape(bl).astype(jnp.int32)
    nsin = jnp.negative(sin_table)
    out = _build(bl)(cos_table, nsin, exp_table, exp_deriv_table, ti, mi)
    return out.reshape(4, b, l)


# speedup vs baseline: 5.4744x; 2.1657x over previous
import functools

import jax
import jax.numpy as jnp
from jax import lax
from jax.experimental import pallas as pl
from jax.experimental.pallas import tpu as pltpu
from jax.experimental.pallas import tpu_sc as plsc

_NC = 2
_NS = 16
_NW = _NC * _NS


def _pick_chunk(per_worker: int) -> int:
    for c in range(min(per_worker // 2, 10240), 0, -8):
        if per_worker % c == 0 and (per_worker // c) % 2 == 0:
            return c
    raise ValueError(f"cannot chunk {per_worker}")


@functools.lru_cache(maxsize=None)
def _build(bl: int):
    assert bl % _NW == 0, bl
    per_w = bl // _NW
    chunk = _pick_chunk(per_w)
    n_steps = per_w // chunk
    assert n_steps >= 4 and n_steps % 2 == 0, n_steps

    mesh = plsc.VectorSubcoreMesh(
        core_axis_name="c", subcore_axis_name="s",
        num_cores=_NC, num_subcores=_NS)

    scratch = []
    for _ in range(2):
        scratch += [pltpu.VMEM((chunk,), jnp.int32)] * 2
        scratch += [pltpu.VMEM((chunk,), jnp.float32)] * 4
    scratch += [pltpu.SemaphoreType.DMA] * 6

    @functools.partial(
        pl.kernel,
        out_type=jax.ShapeDtypeStruct((4 * bl,), jnp.float32),
        mesh=mesh,
        scratch_types=scratch,
    )
    def gather_kernel(cos_h, nsin_h, exp_h, expd_h, ti_h, mi_h, out_h,
                      it0, im0, a0, a1, a2, a3,
                      it1, im1, b0, b1, b2, b3,
                      isem0, gsem0, ssem0, isem1, gsem1, ssem1):
        wid = lax.axis_index("s") * _NC + lax.axis_index("c")
        w_base = wid * per_w
        tabs = (cos_h, nsin_h, exp_h, expd_h)
        bufs = ((it0, im0, (a0, a1, a2, a3), isem0, gsem0, ssem0),
                (it1, im1, (b0, b1, b2, b3), isem1, gsem1, ssem1))

        def fire_idx(s, b):
            it_v, im_v, _, isem, _, _ = bufs[b]
            base = w_base + s * chunk
            pltpu.async_copy(ti_h.at[pl.ds(base, chunk)], it_v, isem)
            pltpu.async_copy(mi_h.at[pl.ds(base, chunk)], im_v, isem)

        def drain_idx(b):
            it_v, im_v, _, isem, _, _ = bufs[b]
            pltpu.make_async_copy(ti_h.at[pl.ds(0, chunk)], it_v, isem).wait()
            pltpu.make_async_copy(mi_h.at[pl.ds(0, chunk)], im_v, isem).wait()

        def fire_gathers(b):
            it_v, im_v, d, _, gsem, _ = bufs[b]
            idxs = (it_v, it_v, im_v, im_v)
            for t in range(4):
                pltpu.async_copy(tabs[t].at[idxs[t]], d[t], gsem)

        def drain_gathers(b):
            _, _, d, _, gsem, _ = bufs[b]
            for t in range(4):
                pltpu.make_async_copy(
                    cos_h.at[pl.ds(0, chunk)], d[t], gsem).wait()

        def fire_stores(s, b):
            _, _, d, _, _, ssem = bufs[b]
            base = w_base + s * chunk
            for t in range(4):
                pltpu.async_copy(d[t], out_h.at[pl.ds(t * bl + base, chunk)],
                                 ssem)

        def drain_stores(b):
            _, _, d, _, _, ssem = bufs[b]
            for t in range(4):
                pltpu.make_async_copy(
                    cos_h.at[pl.ds(0, chunk)], d[t], ssem).wait()

        def body(s, b):
            drain_idx(b)
            drain_stores(b)
            fire_gathers(b)
            drain_gathers(1 - b)
            fire_stores(s - 1, 1 - b)
            return s + 1

        fire_idx(0, 0)
        fire_idx(1, 1)
        drain_idx(0)
        fire_gathers(0)
        drain_idx(1)
        fire_gathers(1)
        drain_gathers(0)
        fire_stores(0, 0)
        fire_idx(2, 0)

        def loop_body(u, c):
            s = 2 * u
            fire_idx(body(s, 0), 1)
            fire_idx(body(s + 1, 1), 0)
            return c
        lax.fori_loop(1, n_steps // 2 - 1, loop_body, 0)

        fire_idx(body(n_steps - 2, 0), 1)
        body(n_steps - 1, 1)

        drain_gathers(1)
        fire_stores(n_steps - 1, 1)
        drain_stores(0)
        drain_stores(1)

    return gather_kernel


def kernel(theta_indices, mag_indices, cos_table, sin_table, exp_table,
           exp_deriv_table):
    b, l = theta_indices.shape
    bl = b * l
    ti = theta_indices.reshape(bl).astype(jnp.int32)
    mi = mag_indices.reshape(bl).astype(jnp.int32)
    nsin = jnp.negative(sin_table)
    out = _build(bl)(cos_table, nsin, exp_table, exp_deriv_table, ti, mi)
    return out.reshape(4, b, l)


# speedup vs baseline: 5.5475x; 1.0134x over previous
import functools

import jax
import jax.numpy as jnp
from jax import lax
from jax.experimental import pallas as pl
from jax.experimental.pallas import tpu as pltpu
from jax.experimental.pallas import tpu_sc as plsc

_NC = 2
_NS = 16
_NW = _NC * _NS


def _pick_chunk(per_worker: int) -> int:
    for c in range(min(per_worker // 2, 10240), 0, -8):
        if per_worker % c == 0 and (per_worker // c) % 2 == 0:
            return c
    raise ValueError(f"cannot chunk {per_worker}")


@functools.lru_cache(maxsize=None)
def _build(bl: int):
    assert bl % _NW == 0, bl
    per_w = bl // _NW
    chunk = _pick_chunk(per_w)
    n_steps = per_w // chunk
    assert n_steps >= 4 and n_steps % 2 == 0, n_steps

    mesh = plsc.VectorSubcoreMesh(
        core_axis_name="c", subcore_axis_name="s",
        num_cores=_NC, num_subcores=_NS)

    scratch = []
    for _ in range(2):
        scratch += [pltpu.VMEM((chunk,), jnp.int32)] * 2
        scratch += [pltpu.VMEM((chunk,), jnp.float32)] * 4
    scratch += [pltpu.SemaphoreType.DMA] * 6

    @functools.partial(
        pl.kernel,
        out_type=jax.ShapeDtypeStruct((4 * bl,), jnp.float32),
        mesh=mesh,
        scratch_types=scratch,
    )
    def gather_kernel(cos_h, nsin_h, exp_h, expd_h, ti_h, mi_h, out_h,
                      it0, im0, a0, a1, a2, a3,
                      it1, im1, b0, b1, b2, b3,
                      isem0, gsem0, ssem0, isem1, gsem1, ssem1):
        wid = lax.axis_index("s") * _NC + lax.axis_index("c")
        w_base = wid * per_w
        tabs = (cos_h, nsin_h, exp_h, expd_h)
        bufs = ((it0, im0, (a0, a1, a2, a3), isem0, gsem0, ssem0),
                (it1, im1, (b0, b1, b2, b3), isem1, gsem1, ssem1))

        def fire_idx(s, b):
            it_v, im_v, _, isem, _, _ = bufs[b]
            base = w_base + s * chunk
            pltpu.async_copy(ti_h.at[pl.ds(base, chunk)], it_v, isem)
            pltpu.async_copy(mi_h.at[pl.ds(base, chunk)], im_v, isem)

        def drain_idx(b):
            it_v, im_v, _, isem, _, _ = bufs[b]
            pltpu.make_async_copy(ti_h.at[pl.ds(0, chunk)], it_v, isem).wait()
            pltpu.make_async_copy(mi_h.at[pl.ds(0, chunk)], im_v, isem).wait()

        half = chunk // 2

        def fire_gathers(b):
            it_v, im_v, d, _, gsem, _ = bufs[b]
            idxs = (it_v, it_v, im_v, im_v)
            for t in range(4):
                for h in range(2):
                    pltpu.async_copy(
                        tabs[t].at[idxs[t].at[pl.ds(h * half, half)]],
                        d[t].at[pl.ds(h * half, half)], gsem)

        def drain_gathers(b):
            _, _, d, _, gsem, _ = bufs[b]
            for t in range(4):
                pltpu.make_async_copy(
                    cos_h.at[pl.ds(0, chunk)], d[t], gsem).wait()

        def fire_stores(s, b):
            _, _, d, _, _, ssem = bufs[b]
            base = w_base + s * chunk
            for t in range(4):
                pltpu.async_copy(d[t], out_h.at[pl.ds(t * bl + base, chunk)],
                                 ssem)

        def drain_stores(b):
            _, _, d, _, _, ssem = bufs[b]
            for t in range(4):
                pltpu.make_async_copy(
                    cos_h.at[pl.ds(0, chunk)], d[t], ssem).wait()

        def body(s, b):
            drain_idx(b)
            drain_stores(b)
            fire_gathers(b)
            drain_gathers(1 - b)
            fire_stores(s - 1, 1 - b)
            return s + 1

        fire_idx(0, 0)
        fire_idx(1, 1)
        drain_idx(0)
        fire_gathers(0)
        drain_idx(1)
        fire_gathers(1)
        drain_gathers(0)
        fire_stores(0, 0)
        fire_idx(2, 0)

        def loop_body(u, c):
            s = 2 * u
            fire_idx(body(s, 0), 1)
            fire_idx(body(s + 1, 1), 0)
            return c
        lax.fori_loop(1, n_steps // 2 - 1, loop_body, 0)

        fire_idx(body(n_steps - 2, 0), 1)
        body(n_steps - 1, 1)

        drain_gathers(1)
        fire_stores(n_steps - 1, 1)
        drain_stores(0)
        drain_stores(1)

    return gather_kernel


def kernel(theta_indices, mag_indices, cos_table, sin_table, exp_table,
           exp_deriv_table):
    b, l = theta_indices.shape
    bl = b * l
    ti = theta_indices.reshape(bl).astype(jnp.int32)
    mi = mag_indices.reshape(bl).astype(jnp.int32)
    nsin = jnp.negative(sin_table)
    out = _build(bl)(cos_table, nsin, exp_table, exp_deriv_table, ti, mi)
    return out.reshape(4, b, l)
